# Initial kernel scaffold; baseline (speedup 1.0000x reference)
import jax
import jax.numpy as jnp
from jax.experimental import pallas as pl


def kernel(idx, target, embedding_table):
    raise NotImplementedError("write your pallas kernel here")


# speedup vs baseline: 1.6823x; 1.6823x over previous
import functools

import jax
import jax.numpy as jnp
from jax import lax
from jax.experimental import pallas as pl
from jax.experimental.pallas import tpu as pltpu
from jax.experimental.pallas import tpu_sc as plsc

VOCAB = 1000
N_TOK = 1024 * 50
NC, NS, L = 2, 16, 16
NW = NC * NS
ROWS_PER_W = N_TOK // NW
CHUNK = 64
NCHUNK = ROWS_PER_W // CHUNK


def _lse_body(table_ref, lse_ref):
    x = table_ref[...]
    m = jnp.max(x, axis=1, keepdims=True)
    s = jnp.sum(jnp.exp(x - m), axis=1, keepdims=True)
    lse_ref[...] = jnp.log(s) + m


_lse_call = pl.pallas_call(
    _lse_body,
    out_shape=jax.ShapeDtypeStruct((VOCAB, 1), jnp.float32),
)


def _sc_body(table_hbm, idx_hbm, tgt_hbm, lse_hbm, out_hbm, psum_hbm,
             idx_v, tgt_v, lse_v, rows_v, acc_v, sem):
    wid = lax.axis_index("s") * NC + lax.axis_index("c")
    base = wid * ROWS_PER_W

    pltpu.sync_copy(idx_hbm.at[wid], idx_v)
    pltpu.sync_copy(tgt_hbm.at[wid], tgt_v)
    pltpu.sync_copy(lse_hbm, lse_v)
    acc_v[...] = jnp.zeros((L,), jnp.float32)

    def chunk_body(k, carry):
        pltpu.async_copy(table_hbm.at[idx_v.at[k]], rows_v, sem).wait()
        pltpu.sync_copy(rows_v, out_hbm.at[pl.ds(base + k * CHUNK, CHUNK)])
        for j in range(CHUNK // L):
            sl = pl.ds(j * L, L)
            idx16 = idx_v[k, sl]
            tgt16 = tgt_v[k, sl]
            lse16 = plsc.load_gather(lse_v, [idx16])
            rid = lax.iota(jnp.int32, L) + j * L
            val16 = plsc.load_gather(rows_v, [rid, tgt16])
            acc_v[...] = acc_v[...] + (lse16 - val16)
        return carry

    lax.fori_loop(0, NCHUNK, chunk_body, 0)
    pltpu.sync_copy(acc_v, psum_hbm.at[wid])


_sc_call = functools.partial(
    pl.kernel,
    mesh=plsc.VectorSubcoreMesh(core_axis_name="c", subcore_axis_name="s"),
    compiler_params=pltpu.CompilerParams(
        use_tc_tiling_on_sc=False, needs_layout_passes=False),
    out_type=[
        jax.ShapeDtypeStruct((N_TOK, VOCAB), jnp.float32),
        jax.ShapeDtypeStruct((NW, L), jnp.float32),
    ],
    scratch_types=[
        pltpu.VMEM((NCHUNK, CHUNK), jnp.int32),
        pltpu.VMEM((NCHUNK, CHUNK), jnp.int32),
        pltpu.VMEM((VOCAB,), jnp.float32),
        pltpu.VMEM((CHUNK, VOCAB), jnp.float32),
        pltpu.VMEM((L,), jnp.float32),
        pltpu.SemaphoreType.DMA,
    ],
)(_sc_body)


def _loss_body(psum_ref, out_ref):
    out_ref[...] = jnp.sum(psum_ref[...], keepdims=True) / N_TOK


_loss_call = pl.pallas_call(
    _loss_body,
    out_shape=jax.ShapeDtypeStruct((1, 1), jnp.float32),
)


def kernel(idx, target, embedding_table):
    idx3 = idx.reshape(NW, NCHUNK, CHUNK).astype(jnp.int32)
    tgt3 = target.reshape(NW, NCHUNK, CHUNK).astype(jnp.int32)
    table = embedding_table.astype(jnp.float32)
    lse = _lse_call(table).reshape(VOCAB)
    logits, psum = _sc_call(table, idx3, tgt3, lse)
    loss = _loss_call(psum).reshape(())
    return logits, loss


# speedup vs baseline: 1.6885x; 1.0037x over previous
import functools

import jax
import jax.numpy as jnp
from jax import lax
from jax.experimental import pallas as pl
from jax.experimental.pallas import tpu as pltpu
from jax.experimental.pallas import tpu_sc as plsc

VOCAB = 1000
N_TOK = 1024 * 50
NC, NS, L = 2, 16, 16
NW = NC * NS
ROWS_PER_W = N_TOK // NW
CHUNK = 32
NCHUNK = ROWS_PER_W // CHUNK
NPAIR = NCHUNK // 2


def _lse_body(table_ref, lse_ref):
    x = table_ref[...]
    m = jnp.max(x, axis=1, keepdims=True)
    s = jnp.sum(jnp.exp(x - m), axis=1, keepdims=True)
    lse_ref[...] = jnp.log(s) + m


_lse_call = pl.pallas_call(
    _lse_body,
    out_shape=jax.ShapeDtypeStruct((VOCAB, 1), jnp.float32),
)


def _sc_body(table_hbm, idx_hbm, tgt_hbm, lse_hbm, out_hbm, psum_hbm,
             idx_v, tgt_v, lse_v, rows0_v, rows1_v, acc_v,
             gsem0, gsem1, ssem0, ssem1):
    wid = lax.axis_index("s") * NC + lax.axis_index("c")
    base = wid * ROWS_PER_W

    pltpu.sync_copy(idx_hbm.at[wid], idx_v)
    pltpu.sync_copy(tgt_hbm.at[wid], tgt_v)
    pltpu.sync_copy(lse_hbm, lse_v)
    acc_v[...] = jnp.zeros((L,), jnp.float32)

    def gather(k, buf, sem):
        return pltpu.async_copy(table_hbm.at[idx_v.at[k]], buf, sem)

    def scatter(k, buf, sem):
        return pltpu.async_copy(buf, out_hbm.at[pl.ds(base + k * CHUNK, CHUNK)], sem)

    def wait_gather(buf, sem):
        pltpu.make_async_copy(table_hbm.at[idx_v.at[0]], buf, sem).wait()

    def wait_scatter(buf, sem):
        pltpu.make_async_copy(buf, out_hbm.at[pl.ds(base, CHUNK)], sem).wait()

    def extract(k, buf):
        for j in range(CHUNK // L):
            sl = pl.ds(j * L, L)
            idx16 = idx_v[k, sl]
            tgt16 = tgt_v[k, sl]
            lse16 = plsc.load_gather(lse_v, [idx16])
            rid = lax.iota(jnp.int32, L) + j * L
            val16 = plsc.load_gather(buf, [rid, tgt16])
            acc_v[...] = acc_v[...] + (lse16 - val16)

    gather(0, rows0_v, gsem0)

    def pair_body(m, carry):
        k0 = 2 * m
        k1 = k0 + 1
        wait_gather(rows0_v, gsem0)

        @pl.when(m > 0)
        def _():
            wait_scatter(rows1_v, ssem1)

        gather(k1, rows1_v, gsem1)
        scatter(k0, rows0_v, ssem0)
        extract(k0, rows0_v)
        wait_gather(rows1_v, gsem1)
        wait_scatter(rows0_v, ssem0)

        @pl.when(m < NPAIR - 1)
        def _():
            gather(k0 + 2, rows0_v, gsem0)

        scatter(k1, rows1_v, ssem1)
        extract(k1, rows1_v)
        return carry

    lax.fori_loop(0, NPAIR, pair_body, 0)
    wait_scatter(rows1_v, ssem1)
    pltpu.sync_copy(acc_v, psum_hbm.at[wid])


_sc_call = functools.partial(
    pl.kernel,
    mesh=plsc.VectorSubcoreMesh(core_axis_name="c", subcore_axis_name="s"),
    compiler_params=pltpu.CompilerParams(
        use_tc_tiling_on_sc=False, needs_layout_passes=False),
    out_type=[
        jax.ShapeDtypeStruct((N_TOK, VOCAB), jnp.float32),
        jax.ShapeDtypeStruct((NW, L), jnp.float32),
    ],
    scratch_types=[
        pltpu.VMEM((NCHUNK, CHUNK), jnp.int32),
        pltpu.VMEM((NCHUNK, CHUNK), jnp.int32),
        pltpu.VMEM((VOCAB,), jnp.float32),
        pltpu.VMEM((CHUNK, VOCAB), jnp.float32),
        pltpu.VMEM((CHUNK, VOCAB), jnp.float32),
        pltpu.VMEM((L,), jnp.float32),
        pltpu.SemaphoreType.DMA,
        pltpu.SemaphoreType.DMA,
        pltpu.SemaphoreType.DMA,
        pltpu.SemaphoreType.DMA,
    ],
)(_sc_body)


def _loss_body(psum_ref, out_ref):
    out_ref[...] = jnp.sum(psum_ref[...], keepdims=True) / N_TOK


_loss_call = pl.pallas_call(
    _loss_body,
    out_shape=jax.ShapeDtypeStruct((1, 1), jnp.float32),
)


def kernel(idx, target, embedding_table):
    idx3 = idx.reshape(NW, NCHUNK, CHUNK).astype(jnp.int32)
    tgt3 = target.reshape(NW, NCHUNK, CHUNK).astype(jnp.int32)
    table = embedding_table.astype(jnp.float32)
    lse = _lse_call(table).reshape(VOCAB)
    logits, psum = _sc_call(table, idx3, tgt3, lse)
    loss = _loss_call(psum).reshape(())
    return logits, loss
